# baseline (speedup 1.0000x reference)
import jax
import jax.numpy as jnp
from jax.experimental import pallas as pl


def _copy_kernel(x_ref, o_ref):
    o_ref[...] = x_ref[...]


def _pallas_copy(x):
    return pl.pallas_call(
        _copy_kernel,
        out_shape=jax.ShapeDtypeStruct(x.shape, x.dtype),
    )(x)


def _mrconv(x_src, x_dst, src_idx, dst_idx, p):
    W, b = p
    diffs = x_src[src_idx] - x_dst[dst_idx]
    agg = jax.ops.segment_max(diffs, dst_idx, num_segments=x_dst.shape[0])
    agg = jnp.where(jnp.isfinite(agg), agg, 0.0)
    return x_dst + jnp.concatenate([x_dst, agg], axis=1) @ W + b


def _sbgcn(f, l, e, v, fl_face, le_loop, le_edge, ev_edge, ev_vert, ff_src, ff_dst, params):
    x_f = jax.nn.relu(f @ params['emb_f'][0] + params['emb_f'][1])
    x_l = jax.nn.relu(l @ params['emb_l'][0] + params['emb_l'][1])
    x_e = jax.nn.relu(e @ params['emb_e'][0] + params['emb_e'][1])
    x_v = jax.nn.relu(v @ params['emb_v'][0] + params['emb_v'][1])
    loop_ids = jnp.arange(x_l.shape[0])
    x_e = _mrconv(x_v, x_e, ev_vert, ev_edge, params['v2e'])
    x_l = _mrconv(x_e, x_l, le_edge, le_loop, params['e2l'])
    x_f = _mrconv(x_l, x_f, loop_ids, fl_face, params['l2f'])
    for _ in range(6):
        x_f = _mrconv(x_f, x_f, ff_src, ff_dst, params['ff'])
    x_l = _mrconv(x_f, x_l, fl_face, loop_ids, params['f2l'])
    x_e = _mrconv(x_l, x_e, le_loop, le_edge, params['l2e'])
    x_v = _mrconv(x_e, x_v, ev_edge, ev_vert, params['e2v'])
    return (x_f, x_e, x_v)


def kernel(f_orig, l_orig, e_orig, v_orig, f_var, l_var, e_var, v_var,
           fl_face, le_loop, le_edge, ev_edge, ev_vert, ff_src, ff_dst, params):
    f_o, e_o, v_o = _sbgcn(f_orig, l_orig, e_orig, v_orig, fl_face, le_loop,
                           le_edge, ev_edge, ev_vert, ff_src, ff_dst, params)
    f_v, e_v, v_v = _sbgcn(f_var, l_var, e_var, v_var, fl_face, le_loop,
                           le_edge, ev_edge, ev_vert, ff_src, ff_dst, params)
    f_o = _pallas_copy(f_o)
    return ((f_o, e_o, v_o), (f_v, e_v, v_v))


# speedup vs baseline: 4.0538x; 4.0538x over previous
import functools

import jax
import jax.numpy as jnp
from jax import lax
from jax.experimental import pallas as pl
from jax.experimental.pallas import tpu as pltpu
from jax.experimental.pallas import tpu_sc as plsc

N_F, N_L, N_E, N_V = 10000, 20000, 30000, 20000
S_FACE, S_LOOP, S_EDGE, S_VERT = 62, 38, 72, 3
EMB = 64
K_FF = 6

NW = 32
RF, RL, RE, RV = 320, 640, 960, 640
NPF, NPL, NPE, NPV = NW * RF, NW * RL, NW * RE, NW * RV

CHUNK = 128

SPF, SPL, SPE, SPV = 64, 40, 72, 8
BN = 1024


@functools.lru_cache(maxsize=None)
def _seg_max_sc(r, e_pad, n_pad_dst):
    mesh = plsc.VectorSubcoreMesh(core_axis_name="c", subcore_axis_name="s")

    @functools.partial(
        pl.kernel,
        out_type=jax.ShapeDtypeStruct((n_pad_dst, EMB), jnp.float32),
        mesh=mesh,
        compiler_params=pltpu.CompilerParams(use_tc_tiling_on_sc=False),
        scratch_types=[
            pltpu.VMEM((CHUNK,), jnp.int32),
            pltpu.VMEM((CHUNK,), jnp.int32),
            pltpu.VMEM((CHUNK,), jnp.int32),
            pltpu.VMEM((CHUNK,), jnp.int32),
            pltpu.VMEM((CHUNK,), jnp.int32),
            pltpu.VMEM((CHUNK,), jnp.int32),
            pltpu.VMEM((CHUNK, EMB), jnp.float32),
            pltpu.VMEM((CHUNK, EMB), jnp.float32),
            pltpu.VMEM((CHUNK, EMB), jnp.float32),
            pltpu.VMEM((r + 1, EMB), jnp.float32),
            pltpu.VMEM((48,), jnp.int32),
            pltpu.SemaphoreType.DMA,
            pltpu.SemaphoreType.DMA,
            pltpu.SemaphoreType.DMA,
            pltpu.SemaphoreType.DMA,
            pltpu.SemaphoreType.DMA,
            pltpu.SemaphoreType.DMA,
            pltpu.SemaphoreType.DMA,
            pltpu.SemaphoreType.DMA,
            pltpu.SemaphoreType.DMA,
        ],
    )
    def kern(xsrc, ssrc, sdst, starts, out,
             si0, si1, si2, di0, di1, di2, rows0, rows1, rows2, acc, stv,
             ssi0, ssi1, ssi2, sdi0, sdi1, sdi2, sr0, sr1, sr2):
        c = lax.axis_index("c")
        s = lax.axis_index("s")
        wid = s * 2 + c
        pltpu.sync_copy(starts, stv)
        svec = stv[pl.ds(wid, 16)]
        start = svec[0]
        end = svec[1]
        lo = wid * r
        base0 = (start // 8) * 8
        n = (end - base0 + CHUNK - 1) // CHUNK

        neg = jnp.full((16,), -jnp.inf, jnp.float32)

        si = (si0, si1, si2)
        di = (di0, di1, di2)
        rows = (rows0, rows1, rows2)
        ssi = (ssi0, ssi1, ssi2)
        sdi = (sdi0, sdi1, sdi2)
        sr = (sr0, sr1, sr2)

        def idx_issue(k, sl):
            base = base0 + k * CHUNK
            pltpu.async_copy(ssrc.at[pl.ds(base, CHUNK)], si[sl], ssi[sl])
            pltpu.async_copy(sdst.at[pl.ds(base, CHUNK)], di[sl], sdi[sl])

        def gather_issue(k, sl):
            base = base0 + k * CHUNK
            pltpu.make_async_copy(
                ssrc.at[pl.ds(base, CHUNK)], si[sl], ssi[sl]).wait()
            pltpu.make_async_copy(
                sdst.at[pl.ds(base, CHUNK)], di[sl], sdi[sl]).wait()
            pltpu.async_copy(xsrc.at[si[sl]], rows[sl], sr[sl])

        def rows_wait(sl):
            pltpu.make_async_copy(xsrc.at[si[sl]], rows[sl], sr[sl]).wait()

        def process(sl):
            rws = rows[sl]
            dis = di[sl]

            def blk_body(bi, carry):
                dvec = dis[pl.ds(bi * 16, 16)]
                for jj in range(16):
                    i = bi * 16 + jj
                    d = dvec[jj]
                    dl = d - lo
                    ok = jnp.logical_and(dl >= 0, dl < r)
                    row = jnp.where(ok, dl, r)
                    for j in range(4):
                        slc = pl.ds(j * 16, 16)
                        acc[row, slc] = jnp.maximum(acc[row, slc], rws[i, slc])
                return carry

            lax.fori_loop(0, CHUNK // 16, blk_body, 0)

        def init_body(q, carry):
            for j in range(4):
                acc[q, pl.ds(j * 16, 16)] = neg
            return carry

        @pl.when(n > 0)
        def _():
            idx_issue(0, 0)

        @pl.when(n > 1)
        def _():
            idx_issue(1, 1)

        lax.fori_loop(0, r + 1, init_body, 0)

        @pl.when(n > 0)
        def _():
            gather_issue(0, 0)

        def step(k, sl):
            @pl.when(k < n)
            def _():
                rows_wait(sl)

                @pl.when(k + 2 < n)
                def _():
                    idx_issue(k + 2, (sl + 2) % 3)

                @pl.when(k + 1 < n)
                def _():
                    gather_issue(k + 1, (sl + 1) % 3)

                process(sl)

        def triple_body(t, carry):
            k0 = t * 3
            step(k0, 0)
            step(k0 + 1, 1)
            step(k0 + 2, 2)
            return carry

        lax.fori_loop(0, (n + 2) // 3, triple_body, 0)

        pltpu.sync_copy(acc.at[pl.ds(0, r)], out.at[pl.ds(lo, r)])

    return kern


def _linear_relu_tc(x, w, b):
    np_, sp = x.shape

    def body(x_ref, w_ref, b_ref, o_ref):
        o_ref[...] = jnp.maximum(
            jnp.dot(x_ref[...], w_ref[...], preferred_element_type=jnp.float32)
            + b_ref[...], 0.0)

    return pl.pallas_call(
        body,
        grid=(np_ // BN,),
        in_specs=[
            pl.BlockSpec((BN, sp), lambda i: (i, 0)),
            pl.BlockSpec((sp, EMB), lambda i: (0, 0)),
            pl.BlockSpec((1, EMB), lambda i: (0, 0)),
        ],
        out_specs=pl.BlockSpec((BN, EMB), lambda i: (i, 0)),
        out_shape=jax.ShapeDtypeStruct((np_, EMB), jnp.float32),
    )(x, w, b)


def _conv_update_tc(xd, m, w1, w2, b):
    np_ = xd.shape[0]

    def body(xd_ref, m_ref, w1_ref, w2_ref, b_ref, o_ref):
        xdv = xd_ref[...]
        mv = m_ref[...]
        agg = jnp.where(mv > -3.0e38, mv - xdv, 0.0)
        o_ref[...] = (
            xdv
            + jnp.dot(xdv, w1_ref[...], preferred_element_type=jnp.float32)
            + jnp.dot(agg, w2_ref[...], preferred_element_type=jnp.float32)
            + b_ref[...])

    return pl.pallas_call(
        body,
        grid=(np_ // BN,),
        in_specs=[
            pl.BlockSpec((BN, EMB), lambda i: (i, 0)),
            pl.BlockSpec((BN, EMB), lambda i: (i, 0)),
            pl.BlockSpec((EMB, EMB), lambda i: (0, 0)),
            pl.BlockSpec((EMB, EMB), lambda i: (0, 0)),
            pl.BlockSpec((1, EMB), lambda i: (0, 0)),
        ],
        out_specs=pl.BlockSpec((BN, EMB), lambda i: (i, 0)),
        out_shape=jax.ShapeDtypeStruct((np_, EMB), jnp.float32),
    )(xd, m, w1, w2, b)


def _finish_pair(ssrc, sdst, e, r):
    e_pad = ((e + 7) // 8) * 8 + 2 * CHUNK
    pad = e_pad - e
    ssrc = jnp.concatenate([ssrc, jnp.zeros((pad,), jnp.int32)])
    sdst = jnp.concatenate([sdst, jnp.full((pad,), 1 << 30, jnp.int32)])
    bounds = jnp.arange(NW + 1, dtype=jnp.int32) * r
    starts = jnp.searchsorted(sdst, bounds, side="left").astype(jnp.int32)
    starts = jnp.concatenate([starts, jnp.zeros((15,), jnp.int32)])
    return ssrc, sdst, starts, e_pad


def _prep_sorted(src, dst, r):
    e = src.shape[0]
    sdst, ssrc = lax.sort((dst, src), num_keys=1)
    return _finish_pair(ssrc, sdst, e, r)


def _prep_identity(src, n_dst, r):
    e = src.shape[0]
    sdst = jnp.arange(n_dst, dtype=jnp.int32)
    return _finish_pair(src, sdst, e, r)


def _pad2(x, rows, cols):
    return jnp.pad(x, ((0, rows - x.shape[0]), (0, cols - x.shape[1])))


def kernel(f_orig, l_orig, e_orig, v_orig, f_var, l_var, e_var, v_var,
           fl_face, le_loop, le_edge, ev_edge, ev_vert, ff_src, ff_dst, params):
    i32 = jnp.int32
    fl_face = fl_face.astype(i32)
    le_loop = le_loop.astype(i32)
    le_edge = le_edge.astype(i32)
    ev_edge = ev_edge.astype(i32)
    ev_vert = ev_vert.astype(i32)
    ff_src = ff_src.astype(i32)
    ff_dst = ff_dst.astype(i32)

    p_v2e = _prep_sorted(ev_vert, ev_edge, RE)
    p_e2l = _prep_sorted(le_edge, le_loop, RL)
    p_l2f = _prep_sorted(jnp.arange(N_L, dtype=i32), fl_face, RF)
    p_ff = _prep_sorted(ff_src, ff_dst, RF)
    p_f2l = _prep_identity(fl_face, N_L, RL)
    p_l2e = _prep_sorted(le_loop, le_edge, RE)
    p_e2v = _prep_sorted(ev_edge, ev_vert, RV)

    def lin(p, sp):
        w, b = p
        return _pad2(w, sp, EMB), b.reshape(1, EMB)

    def conv_w(p):
        w, b = p
        return w[:EMB], w[EMB:], b.reshape(1, EMB)

    ef_w, ef_b = lin(params["emb_f"], SPF)
    el_w, el_b = lin(params["emb_l"], SPL)
    ee_w, ee_b = lin(params["emb_e"], SPE)
    ev_w, ev_b = lin(params["emb_v"], SPV)
    cw = {k: conv_w(params[k]) for k in
          ("v2e", "e2l", "l2f", "ff", "f2l", "l2e", "e2v")}

    def conv(x_src, x_dst, pair, r, n_pad_dst, key):
        ssrc, sdst, starts, e_pad = pair
        m = _seg_max_sc(r, e_pad, n_pad_dst)(x_src, ssrc, sdst, starts)
        w1, w2, b = cw[key]
        return _conv_update_tc(x_dst, m, w1, w2, b)

    def run(f, l, e, v):
        xf = _linear_relu_tc(_pad2(f, NPF, SPF), ef_w, ef_b)
        xl = _linear_relu_tc(_pad2(l, NPL, SPL), el_w, el_b)
        xe = _linear_relu_tc(_pad2(e, NPE, SPE), ee_w, ee_b)
        xv = _linear_relu_tc(_pad2(v, NPV, SPV), ev_w, ev_b)
        xe = conv(xv, xe, p_v2e, RE, NPE, "v2e")
        xl = conv(xe, xl, p_e2l, RL, NPL, "e2l")
        xf = conv(xl, xf, p_l2f, RF, NPF, "l2f")
        for _ in range(K_FF):
            xf = conv(xf, xf, p_ff, RF, NPF, "ff")
        xl = conv(xf, xl, p_f2l, RL, NPL, "f2l")
        xe = conv(xl, xe, p_l2e, RE, NPE, "l2e")
        xv = conv(xe, xv, p_e2v, RV, NPV, "e2v")
        return xf[:N_F], xe[:N_E], xv[:N_V]

    f_o, e_o, v_o = run(f_orig, l_orig, e_orig, v_orig)
    f_v, e_v, v_v = run(f_var, l_var, e_var, v_var)
    return ((f_o, e_o, v_o), (f_v, e_v, v_v))
